# baseline (speedup 1.0000x reference)
import functools

import jax
import jax.numpy as jnp
from jax import lax
from jax.experimental import pallas as pl
from jax.experimental.pallas import tpu as pltpu
from jax.experimental.pallas import tpu_sc as plsc

H = 768
S = 512
EPS = 1e-12

_NC = 2
_NS = 16
_NW = _NC * _NS


def _sc_gather(word_emb, ids_flat, text_len, s_lo, sh, with_avg):
    nb = text_len.shape[0]
    bpw = nb // _NW
    g = 64
    mesh = plsc.VectorSubcoreMesh(core_axis_name="c", subcore_axis_name="s",
                                  num_cores=_NC, num_subcores=_NS)
    out_type = [jax.ShapeDtypeStruct((nb * sh, H), jnp.float32)]
    if with_avg:
        out_type.append(jax.ShapeDtypeStruct((nb, H), jnp.float32))

    @functools.partial(
        pl.kernel,
        out_type=tuple(out_type),
        mesh=mesh,
        scratch_types=[
            pltpu.VMEM((bpw * sh,), jnp.int32),
            pltpu.VMEM((2, g, H), jnp.float32),
            pltpu.VMEM((16,), jnp.int32),
            pltpu.VMEM((16,), jnp.int32),
            pltpu.VMEM((16, H), jnp.float32),
            pltpu.VMEM((16,), jnp.int32),
            pltpu.VMEM((H,), jnp.float32),
            pltpu.SemaphoreType.DMA,
            pltpu.SemaphoreType.DMA,
        ],
    )
    def gather_kernel(table_hbm, idx_hbm, tl_hbm, *outs_and_scratch):
        if with_avg:
            (out_hbm, avg_hbm, idxall_v, rows2_v, wids_v, widx_v, wrows_v,
             tl_v, avg_v, sem, sem_w) = outs_and_scratch
        else:
            (out_hbm, idxall_v, rows2_v, wids_v, widx_v, wrows_v,
             tl_v, avg_v, sem, sem_w) = outs_and_scratch
        wid = lax.axis_index("s") * _NC + lax.axis_index("c")

        iota = lax.broadcasted_iota(jnp.int32, (16,), 0)
        pltpu.sync_copy(tl_hbm.at[pl.ds(wid * bpw, bpw)], tl_v.at[pl.ds(0, bpw)])
        tl_vec = tl_v[...]

        for j in range(bpw):
            ln_j = tl_vec[j]
            nch = (jnp.clip(ln_j - 1 - s_lo, 0, sh) + (g - 1)) // g
            b_j = wid * bpw + j
            base_out = b_j * sh
            loc_b = j * sh

            @pl.when(nch > 0)
            def _():
                pltpu.sync_copy(idx_hbm.at[pl.ds(b_j * S + s_lo, sh)],
                                idxall_v.at[pl.ds(loc_b, sh)])

            def body(i, carry):
                cur = i % 2
                off = pl.multiple_of(base_out + i * g, g)
                loff = pl.multiple_of(loc_b + i * g, g)

                @pl.when(i >= 2)
                def _():
                    pltpu.make_async_copy(
                        rows2_v.at[cur], out_hbm.at[pl.ds(off, g)],
                        sem_w).wait()

                pltpu.async_copy(
                    table_hbm.at[idxall_v.at[pl.ds(loff, g)]],
                    rows2_v.at[cur], sem).wait()
                pltpu.make_async_copy(
                    rows2_v.at[cur], out_hbm.at[pl.ds(off, g)], sem_w).start()
                return carry

            lax.fori_loop(0, nch, body, 0)

            for d in range(2):
                @pl.when(nch >= d + 1)
                def _():
                    pltpu.make_async_copy(
                        rows2_v.at[0], out_hbm.at[pl.ds(base_out, g)],
                        sem_w).wait()

        if not with_avg:
            return

        for j in range(bpw):
            b = wid * bpw + j
            ln = tl_vec[j]
            c = jnp.maximum(ln - 6, 0)
            c8 = jnp.minimum((c // 8) * 8, S - 16)
            pltpu.sync_copy(idx_hbm.at[pl.ds(b * S + c8, 16)], wids_v)
            gidx = jnp.minimum((c - c8) + iota, 15)
            widx_v[...] = wids_v[...].at[gidx].get(mode="promise_in_bounds")
            pltpu.async_copy(table_hbm.at[widx_v], wrows_v, sem).wait()
            nlast = ln - 2 - c

            def kbody(k, carry):
                koff = pl.multiple_of(k * 16, 16)
                acc = jnp.zeros((16,), jnp.float32)
                for j2 in range(5):
                    w = jnp.where(nlast >= j2, 0.2, 0.0)
                    acc = acc + wrows_v[j2, pl.ds(koff, 16)] * w
                avg_v[pl.ds(koff, 16)] = acc
                return carry

            lax.fori_loop(0, H // 16, kbody, 0)
            pltpu.sync_copy(avg_v, avg_hbm.at[b])

    return gather_kernel(word_emb, ids_flat, text_len)


def _tc_combine(we_q, avg, text_len, pe_q, consts, s_lo, sh, prev_out):
    nb = text_len.shape[0]
    grid = (nb,)
    qs = S // sh
    qi = s_lo // sh
    nck = sh // 64
    ck = 64

    def body(tl_ref, we_ref, avg_ref, pe_ref, c_ref, *rest):
        if prev_out is None:
            out_ref, web_ref, sem = rest
        else:
            _prev_ref, out_ref, web_ref, sem = rest
        b = pl.program_id(0)
        ln = tl_ref[b]

        def chunk_copies(bb, buf, do_start):
            lnb = tl_ref[bb]
            for k in range(nck):
                @pl.when(s_lo + k * ck <= lnb - 2)
                def _():
                    cp = pltpu.make_async_copy(
                        we_ref.at[pl.ds(bb * sh + k * ck, ck)],
                        buf.at[pl.ds(k * ck, ck)], sem)
                    if do_start:
                        cp.start()
                    else:
                        cp.wait()

        @pl.when(b == 0)
        def _():
            chunk_copies(0, web_ref.at[0], True)

        nxt = jnp.minimum(b + 1, nb - 1)

        @pl.when(b + 1 < nb)
        def _():
            chunk_copies(nxt, web_ref.at[(b + 1) % 2], True)

        chunk_copies(b, web_ref.at[b % 2], False)

        svec = s_lo + lax.broadcasted_iota(jnp.int32, (sh, 1), 0)
        sep = c_ref[0:1, :]
        pad = c_ref[1:2, :]
        gamma = c_ref[2:3, :]
        beta = c_ref[3:4, :]
        sel = jnp.where(svec < ln - 6, web_ref[b % 2],
                        jnp.where(svec == ln - 6, avg_ref[0],
                                  jnp.where(svec == ln - 5, sep, pad)))
        x = sel + pe_ref[...]
        mu = jnp.mean(x, axis=1, keepdims=True)
        xc = x - mu
        var = jnp.mean(xc * xc, axis=1, keepdims=True)
        y = xc * lax.rsqrt(var + EPS)
        out_ref[...] = y * gamma + beta

    in_specs = [
        pl.BlockSpec(memory_space=pltpu.SMEM),
        pl.BlockSpec(memory_space=pl.ANY),
        pl.BlockSpec((1, 1, H), lambda b: (b, 0, 0)),
        pl.BlockSpec((sh, H), lambda b: (0, 0)),
        pl.BlockSpec((8, H), lambda b: (0, 0)),
    ]
    args = [text_len, we_q, avg.reshape(-1, 1, H), pe_q, consts]
    aliases = {}
    if prev_out is not None:
        in_specs.append(pl.BlockSpec(memory_space=pl.ANY))
        args.append(prev_out)
        aliases = {5: 0}
    return pl.pallas_call(
        body,
        grid=grid,
        in_specs=in_specs,
        out_specs=pl.BlockSpec((sh, H), lambda b: (b * qs + qi, 0)),
        out_shape=jax.ShapeDtypeStruct((nb * S, H), jnp.float32),
        scratch_shapes=[
            pltpu.VMEM((2, sh, H), jnp.float32),
            pltpu.SemaphoreType.DMA,
        ],
        input_output_aliases=aliases,
    )(*args)


def kernel(input_ids, text_len, word_emb, pos_emb, type_emb, ln_gamma, ln_beta):
    b, s = input_ids.shape
    ids_flat = input_ids.reshape(-1).astype(jnp.int32)
    tl = text_len.astype(jnp.int32)
    pe_plus = pos_emb + type_emb[0][None, :]
    consts = jnp.concatenate(
        [word_emb[102:103], word_emb[0:1], ln_gamma[None, :], ln_beta[None, :],
         jnp.zeros((4, H), jnp.float32)], axis=0)
    sh = S // 2
    we0, avg = _sc_gather(word_emb, ids_flat, tl, 0, sh, True)
    (we1,) = _sc_gather(word_emb, ids_flat, tl, sh, sh, False)
    out = _tc_combine(we0, avg, tl, pe_plus[:sh], consts, 0, sh, None)
    out = _tc_combine(we1, avg, tl, pe_plus[sh:], consts, sh, sh, out)
    return out.reshape(b, s, H)


# speedup vs baseline: 1.1972x; 1.1972x over previous
import functools

import jax
import jax.numpy as jnp
from jax import lax
from jax.experimental import pallas as pl
from jax.experimental.pallas import tpu as pltpu
from jax.experimental.pallas import tpu_sc as plsc

H = 768
S = 512
EPS = 1e-12

_NC = 2
_NS = 16
_NW = _NC * _NS


def _sc_gather(word_emb, ids_flat, text_len):
    n = ids_flat.shape[0]
    nb = text_len.shape[0]
    rpw = n // _NW
    bpw = nb // _NW
    g = 64
    mesh = plsc.VectorSubcoreMesh(core_axis_name="c", subcore_axis_name="s",
                                  num_cores=_NC, num_subcores=_NS)

    @functools.partial(
        pl.kernel,
        out_type=(jax.ShapeDtypeStruct((n, H), jnp.float32),
                  jax.ShapeDtypeStruct((nb, H), jnp.float32)),
        mesh=mesh,
        scratch_types=[
            pltpu.VMEM((S * (nb // _NW),), jnp.int32),
            pltpu.VMEM((2, g, H), jnp.float32),
            pltpu.VMEM((16,), jnp.int32),
            pltpu.VMEM((16,), jnp.int32),
            pltpu.VMEM((16, H), jnp.float32),
            pltpu.VMEM((16,), jnp.int32),
            pltpu.VMEM((H,), jnp.float32),
            pltpu.SemaphoreType.DMA,
            pltpu.SemaphoreType.DMA,
        ],
    )
    def gather_kernel(table_hbm, idx_hbm, tl_hbm, out_hbm, avg_hbm,
                      idxall_v, rows2_v, wids_v, widx_v, wrows_v, tl_v, avg_v,
                      sem, sem_w):
        wid = lax.axis_index("s") * _NC + lax.axis_index("c")

        iota = lax.broadcasted_iota(jnp.int32, (16,), 0)
        pltpu.sync_copy(tl_hbm.at[pl.ds(wid * bpw, bpw)], tl_v.at[pl.ds(0, bpw)])
        tl_vec = tl_v[...]
        base_w = wid * rpw
        pltpu.sync_copy(idx_hbm.at[pl.ds(base_w, rpw)], idxall_v)

        for j in range(bpw):
            ln_j = tl_vec[j]
            nch = (jnp.clip(ln_j - 1, 0, S) + (g - 1)) // g
            base_b = (wid * bpw + j) * S
            loc_b = j * S

            def body(i, carry):
                cur = i % 2
                off = pl.multiple_of(base_b + i * g, g)
                loff = pl.multiple_of(loc_b + i * g, g)

                @pl.when(i >= 2)
                def _():
                    pltpu.make_async_copy(
                        rows2_v.at[cur], out_hbm.at[pl.ds(off, g)],
                        sem_w).wait()

                pltpu.async_copy(
                    table_hbm.at[idxall_v.at[pl.ds(loff, g)]],
                    rows2_v.at[cur], sem).wait()
                pltpu.make_async_copy(
                    rows2_v.at[cur], out_hbm.at[pl.ds(off, g)], sem_w).start()
                return carry

            lax.fori_loop(0, nch, body, 0)

            for d in range(2):
                @pl.when(nch >= d + 1)
                def _():
                    pltpu.make_async_copy(
                        rows2_v.at[0], out_hbm.at[pl.ds(base_b, g)],
                        sem_w).wait()

        for j in range(bpw):
            b = wid * bpw + j
            ln = tl_vec[j]
            c = jnp.maximum(ln - 6, 0)
            c8 = jnp.minimum((c // 8) * 8, S - 16)
            pltpu.sync_copy(idx_hbm.at[pl.ds(b * S + c8, 16)], wids_v)
            gidx = jnp.minimum((c - c8) + iota, 15)
            widx_v[...] = wids_v[...].at[gidx].get(mode="promise_in_bounds")
            pltpu.async_copy(table_hbm.at[widx_v], wrows_v, sem).wait()
            nlast = ln - 2 - c

            def kbody(k, carry):
                koff = pl.multiple_of(k * 16, 16)
                acc = jnp.zeros((16,), jnp.float32)
                for j2 in range(5):
                    w = jnp.where(nlast >= j2, 0.2, 0.0)
                    acc = acc + wrows_v[j2, pl.ds(koff, 16)] * w
                avg_v[pl.ds(koff, 16)] = acc
                return carry

            lax.fori_loop(0, H // 16, kbody, 0)
            pltpu.sync_copy(avg_v, avg_hbm.at[b])

    return gather_kernel(word_emb, ids_flat, text_len)


def _tc_combine(we_flat, avg, text_len, pe_plus, consts):
    n = we_flat.shape[0]
    blk = S
    nb = n // S
    grid = (nb,)

    nck = 8
    ck = S // nck

    def body(tl_ref, we_ref, avg_ref, pe_ref, c_ref, out_ref, web_ref, sem):
        b = pl.program_id(0)
        ln = tl_ref[b]

        def chunk_copies(bb, buf, do_start):
            lnb = tl_ref[bb]
            for k in range(nck):
                @pl.when(k * ck <= lnb - 2)
                def _():
                    cp = pltpu.make_async_copy(
                        we_ref.at[pl.ds(bb * S + k * ck, ck)],
                        buf.at[pl.ds(k * ck, ck)], sem)
                    if do_start:
                        cp.start()
                    else:
                        cp.wait()

        @pl.when(b == 0)
        def _():
            chunk_copies(0, web_ref.at[0], True)

        nxt = jnp.minimum(b + 1, nb - 1)

        @pl.when(b + 1 < nb)
        def _():
            chunk_copies(nxt, web_ref.at[(b + 1) % 2], True)

        chunk_copies(b, web_ref.at[b % 2], False)

        svec = lax.broadcasted_iota(jnp.int32, (blk, 1), 0)
        sep = c_ref[0:1, :]
        pad = c_ref[1:2, :]
        gamma = c_ref[2:3, :]
        beta = c_ref[3:4, :]
        sel = jnp.where(svec < ln - 6, web_ref[b % 2],
                        jnp.where(svec == ln - 6, avg_ref[0],
                                  jnp.where(svec == ln - 5, sep, pad)))
        x = sel + pe_ref[...]
        mu = jnp.mean(x, axis=1, keepdims=True)
        xc = x - mu
        var = jnp.mean(xc * xc, axis=1, keepdims=True)
        y = xc * lax.rsqrt(var + EPS)
        out_ref[...] = y * gamma + beta

    return pl.pallas_call(
        body,
        grid=grid,
        in_specs=[
            pl.BlockSpec(memory_space=pltpu.SMEM),
            pl.BlockSpec(memory_space=pl.ANY),
            pl.BlockSpec((1, 1, H), lambda b: (b, 0, 0)),
            pl.BlockSpec((blk, H), lambda b: (0, 0)),
            pl.BlockSpec((8, H), lambda b: (0, 0)),
        ],
        out_specs=pl.BlockSpec((blk, H), lambda b: (b, 0)),
        out_shape=jax.ShapeDtypeStruct((n, H), jnp.float32),
        scratch_shapes=[
            pltpu.VMEM((2, S, H), jnp.float32),
            pltpu.SemaphoreType.DMA,
        ],
    )(text_len, we_flat, avg.reshape(-1, 1, H), pe_plus, consts)


def kernel(input_ids, text_len, word_emb, pos_emb, type_emb, ln_gamma, ln_beta):
    b, s = input_ids.shape
    ids_flat = input_ids.reshape(-1).astype(jnp.int32)
    tl = text_len.astype(jnp.int32)
    pe_plus = pos_emb + type_emb[0][None, :]
    consts = jnp.concatenate(
        [word_emb[102:103], word_emb[0:1], ln_gamma[None, :], ln_beta[None, :],
         jnp.zeros((4, H), jnp.float32)], axis=0)
    we_flat, avg = _sc_gather(word_emb, ids_flat, tl)
    out = _tc_combine(we_flat, avg, tl, pe_plus, consts)
    return out.reshape(b, s, H)


# speedup vs baseline: 1.2076x; 1.0087x over previous
import functools

import jax
import jax.numpy as jnp
from jax import lax
from jax.experimental import pallas as pl
from jax.experimental.pallas import tpu as pltpu
from jax.experimental.pallas import tpu_sc as plsc

H = 768
S = 512
EPS = 1e-12

_NC = 2
_NS = 16
_NW = _NC * _NS


def _sc_gather(word_emb, ids_flat, text_len, perm, tl_perm):
    n = ids_flat.shape[0]
    nb = text_len.shape[0]
    rpw = n // _NW
    bpw = nb // _NW
    g = 64
    mesh = plsc.VectorSubcoreMesh(core_axis_name="c", subcore_axis_name="s",
                                  num_cores=_NC, num_subcores=_NS)

    @functools.partial(
        pl.kernel,
        out_type=(jax.ShapeDtypeStruct((n, H), jnp.float32),
                  jax.ShapeDtypeStruct((nb, H), jnp.float32)),
        mesh=mesh,
        scratch_types=[
            pltpu.VMEM((S * (nb // _NW),), jnp.int32),
            pltpu.VMEM((2, g, H), jnp.float32),
            pltpu.VMEM((16,), jnp.int32),
            pltpu.VMEM((16,), jnp.int32),
            pltpu.VMEM((16, H), jnp.float32),
            pltpu.VMEM((16,), jnp.int32),
            pltpu.VMEM((16,), jnp.int32),
            pltpu.VMEM((H,), jnp.float32),
            pltpu.SemaphoreType.DMA,
            pltpu.SemaphoreType.DMA,
        ],
    )
    def gather_kernel(table_hbm, idx_hbm, tl_hbm, pm_hbm, out_hbm, avg_hbm,
                      idxall_v, rows2_v, wids_v, widx_v, wrows_v, tl_v, pm_v,
                      avg_v, sem, sem_w):
        wid = lax.axis_index("s") * _NC + lax.axis_index("c")

        iota = lax.broadcasted_iota(jnp.int32, (16,), 0)
        pltpu.sync_copy(tl_hbm.at[pl.ds(wid * bpw, bpw)], tl_v.at[pl.ds(0, bpw)])
        pltpu.sync_copy(pm_hbm.at[pl.ds(wid * bpw, bpw)], pm_v.at[pl.ds(0, bpw)])
        tl_vec = tl_v[...]
        pm_vec = pm_v[...]

        for j in range(bpw):
            ln_j = tl_vec[j]
            nch = (jnp.clip(ln_j - 1, 0, S) + (g - 1)) // g
            base_b = pm_vec[j] * S
            loc_b = j * S

            @pl.when(nch > 0)
            def _():
                pltpu.sync_copy(idx_hbm.at[pl.ds(base_b, S)],
                                idxall_v.at[pl.ds(loc_b, S)])

            def body(i, carry):
                cur = i % 2
                off = pl.multiple_of(base_b + i * g, g)
                loff = pl.multiple_of(loc_b + i * g, g)

                @pl.when(i >= 2)
                def _():
                    pltpu.make_async_copy(
                        rows2_v.at[cur], out_hbm.at[pl.ds(off, g)],
                        sem_w).wait()

                pltpu.async_copy(
                    table_hbm.at[idxall_v.at[pl.ds(loff, g)]],
                    rows2_v.at[cur], sem).wait()
                pltpu.make_async_copy(
                    rows2_v.at[cur], out_hbm.at[pl.ds(off, g)], sem_w).start()
                return carry

            lax.fori_loop(0, nch, body, 0)

            for d in range(2):
                @pl.when(nch >= d + 1)
                def _():
                    pltpu.make_async_copy(
                        rows2_v.at[0], out_hbm.at[pl.ds(base_b, g)],
                        sem_w).wait()

        for j in range(bpw):
            b = pm_vec[j]
            ln = tl_vec[j]
            c = jnp.maximum(ln - 6, 0)
            c8 = jnp.minimum((c // 8) * 8, S - 16)
            pltpu.sync_copy(idx_hbm.at[pl.ds(b * S + c8, 16)], wids_v)
            gidx = jnp.minimum((c - c8) + iota, 15)
            widx_v[...] = wids_v[...].at[gidx].get(mode="promise_in_bounds")
            pltpu.async_copy(table_hbm.at[widx_v], wrows_v, sem).wait()
            nlast = ln - 2 - c

            def kbody(k, carry):
                koff = pl.multiple_of(k * 16, 16)
                acc = jnp.zeros((16,), jnp.float32)
                for j2 in range(5):
                    w = jnp.where(nlast >= j2, 0.2, 0.0)
                    acc = acc + wrows_v[j2, pl.ds(koff, 16)] * w
                avg_v[pl.ds(koff, 16)] = acc
                return carry

            lax.fori_loop(0, H // 16, kbody, 0)
            pltpu.sync_copy(avg_v, avg_hbm.at[b])

    return gather_kernel(word_emb, ids_flat, tl_perm, perm)


def _tc_combine(we_flat, avg, text_len, pe_plus, consts):
    n = we_flat.shape[0]
    blk = S
    nb = n // S
    grid = (nb,)

    nck = 8
    ck = S // nck

    def body(tl_ref, we_ref, avg_ref, pe_ref, c_ref, out_ref, web_ref, sem):
        b = pl.program_id(0)
        ln = tl_ref[b]

        def chunk_copies(bb, buf, do_start):
            lnb = tl_ref[bb]
            for k in range(nck):
                @pl.when(k * ck <= lnb - 2)
                def _():
                    cp = pltpu.make_async_copy(
                        we_ref.at[pl.ds(bb * S + k * ck, ck)],
                        buf.at[pl.ds(k * ck, ck)], sem)
                    if do_start:
                        cp.start()
                    else:
                        cp.wait()

        @pl.when(b == 0)
        def _():
            chunk_copies(0, web_ref.at[0], True)

        nxt = jnp.minimum(b + 1, nb - 1)

        @pl.when(b + 1 < nb)
        def _():
            chunk_copies(nxt, web_ref.at[(b + 1) % 2], True)

        chunk_copies(b, web_ref.at[b % 2], False)

        svec = lax.broadcasted_iota(jnp.int32, (blk, 1), 0)
        sep = c_ref[0:1, :]
        pad = c_ref[1:2, :]
        gamma = c_ref[2:3, :]
        beta = c_ref[3:4, :]
        sel = jnp.where(svec < ln - 6, web_ref[b % 2],
                        jnp.where(svec == ln - 6, avg_ref[0],
                                  jnp.where(svec == ln - 5, sep, pad)))
        x = sel + pe_ref[...]
        mu = jnp.mean(x, axis=1, keepdims=True)
        xc = x - mu
        var = jnp.mean(xc * xc, axis=1, keepdims=True)
        y = xc * lax.rsqrt(var + EPS)
        out_ref[...] = y * gamma + beta

    return pl.pallas_call(
        body,
        grid=grid,
        in_specs=[
            pl.BlockSpec(memory_space=pltpu.SMEM),
            pl.BlockSpec(memory_space=pl.ANY),
            pl.BlockSpec((1, 1, H), lambda b: (b, 0, 0)),
            pl.BlockSpec((blk, H), lambda b: (0, 0)),
            pl.BlockSpec((8, H), lambda b: (0, 0)),
        ],
        out_specs=pl.BlockSpec((blk, H), lambda b: (b, 0)),
        out_shape=jax.ShapeDtypeStruct((n, H), jnp.float32),
        scratch_shapes=[
            pltpu.VMEM((2, S, H), jnp.float32),
            pltpu.SemaphoreType.DMA,
        ],
    )(text_len, we_flat, avg.reshape(-1, 1, H), pe_plus, consts)


def kernel(input_ids, text_len, word_emb, pos_emb, type_emb, ln_gamma, ln_beta):
    b, s = input_ids.shape
    ids_flat = input_ids.reshape(-1).astype(jnp.int32)
    tl = text_len.astype(jnp.int32)
    pe_plus = pos_emb + type_emb[0][None, :]
    consts = jnp.concatenate(
        [word_emb[102:103], word_emb[0:1], ln_gamma[None, :], ln_beta[None, :],
         jnp.zeros((4, H), jnp.float32)], axis=0)
    bpw = b // _NW
    order = jnp.argsort(-tl)
    slots = jnp.arange(bpw)[None, :]
    w = jnp.arange(_NW)[:, None]
    sp = slots * _NW + jnp.where(slots % 2 == 0, w, _NW - 1 - w)
    perm = order[sp.reshape(-1)].astype(jnp.int32)
    tl_perm = tl[perm]
    we_flat, avg = _sc_gather(word_emb, ids_flat, tl, perm, tl_perm)
    out = _tc_combine(we_flat, avg, tl, pe_plus, consts)
    return out.reshape(b, s, H)
